# Initial kernel scaffold; baseline (speedup 1.0000x reference)
import jax
import jax.numpy as jnp
from jax.experimental import pallas as pl


def kernel(emb, graph, gcn_W1, gcn_b1, gcn_W2, gcn_b2, ln_a, ln_b, ff_W1, ff_b1, ff_W2, ff_b2):
    raise NotImplementedError("write your pallas kernel here")


# speedup vs baseline: 1.2759x; 1.2759x over previous
import functools

import jax
import jax.numpy as jnp
from jax.experimental import pallas as pl
from jax.experimental.pallas import tpu as pltpu

B, N, D, H = 4, 2048, 128, 128
TILE = 512


def _gcn_kernel(graph_ref, emb_ref, w1_ref, b1_ref, w2_ref, b2_ref,
                x2_ref, x1_scratch):
    t = pl.program_id(1)

    @pl.when(t == 0)
    def _():
        x1_scratch[...] = (
            jnp.dot(emb_ref[0], w1_ref[...],
                    preferred_element_type=jnp.float32) + b1_ref[...]
        )

    h = jnp.dot(graph_ref[0], x1_scratch[...],
                preferred_element_type=jnp.float32)
    h = jnp.maximum(h, 0.0)
    x2_ref[0] = jnp.dot(h, w2_ref[...],
                        preferred_element_type=jnp.float32) + b2_ref[...]


def _fused_kernel(graph_ref, x2_ref, emb_ref, ln_a_ref, ln_b_ref,
                  fw1_ref, fb1_ref, fw2_ref, fb2_ref, out_ref):
    eps = 1e-6
    temp = jnp.dot(graph_ref[0], x2_ref[0],
                   preferred_element_type=jnp.float32)
    mean = jnp.mean(temp, axis=-1, keepdims=True)
    cent = temp - mean
    var = jnp.sum(cent * cent, axis=-1, keepdims=True) / (D - 1)
    std = jnp.sqrt(var)
    normed = ln_a_ref[...] * cent / (std + eps) + ln_b_ref[...]
    num_fea = normed + emb_ref[0]
    ff = jnp.dot(num_fea, fw1_ref[...],
                 preferred_element_type=jnp.float32) + fb1_ref[...]
    ff = jnp.maximum(ff, 0.0)
    ff = jnp.dot(ff, fw2_ref[...],
                 preferred_element_type=jnp.float32) + fb2_ref[...]
    out_ref[0] = ff + num_fea


@jax.jit
def kernel(emb, graph, gcn_W1, gcn_b1, gcn_W2, gcn_b2, ln_a, ln_b,
           ff_W1, ff_b1, ff_W2, ff_b2):
    grid = (B, N // TILE)
    graph_spec = pl.BlockSpec((1, TILE, N), lambda b, t: (b, t, 0))
    row_spec = pl.BlockSpec((1, TILE, D), lambda b, t: (b, t, 0))
    full_spec = pl.BlockSpec((1, N, D), lambda b, t: (b, 0, 0))
    mat_spec = pl.BlockSpec((D, H), lambda b, t: (0, 0))
    vec_spec = pl.BlockSpec((H,), lambda b, t: (0,))

    x2 = pl.pallas_call(
        _gcn_kernel,
        grid=grid,
        in_specs=[graph_spec, full_spec, mat_spec, vec_spec,
                  pl.BlockSpec((H, D), lambda b, t: (0, 0)),
                  pl.BlockSpec((D,), lambda b, t: (0,))],
        out_specs=row_spec,
        out_shape=jax.ShapeDtypeStruct((B, N, D), jnp.float32),
        scratch_shapes=[pltpu.VMEM((N, H), jnp.float32)],
    )(graph, emb, gcn_W1, gcn_b1, gcn_W2, gcn_b2)

    out = pl.pallas_call(
        _fused_kernel,
        grid=grid,
        in_specs=[graph_spec, full_spec, row_spec,
                  pl.BlockSpec((D,), lambda b, t: (0,)),
                  pl.BlockSpec((D,), lambda b, t: (0,)),
                  mat_spec, vec_spec,
                  pl.BlockSpec((H, D), lambda b, t: (0, 0)),
                  pl.BlockSpec((D,), lambda b, t: (0,))],
        out_specs=row_spec,
        out_shape=jax.ShapeDtypeStruct((B, N, D), jnp.float32),
    )(graph, x2, emb, ln_a, ln_b, ff_W1, ff_b1, ff_W2, ff_b2)
    return out
